# baseline (speedup 1.0000x reference)
import functools

import jax
import jax.numpy as jnp
from jax import lax
from jax.experimental import pallas as pl
from jax.experimental.pallas import tpu as pltpu
from jax.experimental.pallas import tpu_sc as plsc

_N = 320000
_D = 128
_H = 256
_S = 10000

_NSLAB = 2
_NSR = _N // _NSLAB


_BN = 2000


def _pn_body(x_ref, w1_ref, b1_ref, w2_ref, b2_ref, o_ref):
    h = jnp.maximum(
        jnp.dot(x_ref[...], w1_ref[...], preferred_element_type=jnp.float32)
        + b1_ref[...],
        0.0,
    )
    o_ref[...] = (
        jnp.dot(h, w2_ref[...], preferred_element_type=jnp.float32) + b2_ref[...]
    )


def _point_net(x, w1, b1, w2, b2):
    n = x.shape[0]
    return pl.pallas_call(
        _pn_body,
        grid=(n // _BN,),
        in_specs=[
            pl.BlockSpec((_BN, _D), lambda i: (i, 0)),
            pl.BlockSpec((_D, _H), lambda i: (0, 0)),
            pl.BlockSpec((1, _H), lambda i: (0, 0)),
            pl.BlockSpec((_H, _D), lambda i: (0, 0)),
            pl.BlockSpec((1, _D), lambda i: (0, 0)),
        ],
        out_specs=pl.BlockSpec((_BN, _D), lambda i: (i, 0)),
        out_shape=jax.ShapeDtypeStruct((n, _D), jnp.float32),
    )(x, w1, b1, w2, b2)


_BS = 2000


def _rn_body(p_ref, q_ref, w1_ref, b1_ref, w2_ref, b2_ref, o_ref):
    seg = (p_ref[0] + p_ref[1]) + (q_ref[0] + q_ref[1])
    h = jnp.maximum(
        jnp.dot(seg, w1_ref[...], preferred_element_type=jnp.float32) + b1_ref[...],
        0.0,
    )
    o_ref[...] = (
        jnp.dot(h, w2_ref[...], preferred_element_type=jnp.float32) + b2_ref[...]
    )


def _reduce_net(p, q, w1, b1, w2, b2):
    return pl.pallas_call(
        _rn_body,
        grid=(_S // _BS,),
        in_specs=[
            pl.BlockSpec((2, _BS, _D), lambda i: (0, i, 0)),
            pl.BlockSpec((2, _BS, _D), lambda i: (0, i, 0)),
            pl.BlockSpec((_D, _H), lambda i: (0, 0)),
            pl.BlockSpec((1, _H), lambda i: (0, 0)),
            pl.BlockSpec((_H, _D), lambda i: (0, 0)),
            pl.BlockSpec((1, _D), lambda i: (0, 0)),
        ],
        out_specs=pl.BlockSpec((_BS, _D), lambda i: (i, 0)),
        out_shape=jax.ShapeDtypeStruct((_S, _D), jnp.float32),
    )(p, q, w1, b1, w2, b2)


_NC = 2
_NS = 16
_NW = _NC * _NS
_RPW = _NSR // _NW
_CH = 40
_NCH = _RPW // _CH
_NBUF = 4
_LEAD = 2
_SP = 10112
_ZR = _SP // _NS


@functools.cache
def _make_seg_sum():
    mesh = plsc.VectorSubcoreMesh(core_axis_name="c", subcore_axis_name="s")

    @functools.partial(
        pl.kernel,
        mesh=mesh,
        out_type=jax.ShapeDtypeStruct((_NC, _SP, _D), jnp.float32),
        scratch_types=(
            [pltpu.VMEM((_CH,), jnp.int32) for _ in range(_NBUF)]
            + [pltpu.VMEM((_CH, _D), jnp.float32) for _ in range(_NBUF)]
            + [pltpu.VMEM_SHARED((_SP, _D), jnp.float32)]
            + [pltpu.SemaphoreType.DMA for _ in range(2 * _NBUF)]
        ),
    )
    def seg_sum(pt_hbm, idx_hbm, zrows_hbm, out_hbm, *scr):
        ibufs = scr[:_NBUF]
        rbufs = scr[_NBUF : 2 * _NBUF]
        seg_sh = scr[2 * _NBUF]
        lsems = scr[2 * _NBUF + 1 : 3 * _NBUF + 1]
        ssems = scr[3 * _NBUF + 1 : 4 * _NBUF + 1]
        c = lax.axis_index("c")
        s = lax.axis_index("s")
        wid = c * _NS + s
        pltpu.sync_copy(zrows_hbm, seg_sh.at[pl.ds(s * _ZR, _ZR)])
        plsc.subcore_barrier()

        base = wid * _RPW

        def rows_src(j):
            return pt_hbm.at[pl.ds(base + j * _CH, _CH)]

        def idx_src(j):
            return idx_hbm.at[pl.ds(base + j * _CH, _CH)]

        def start_loads(j):
            b = j % _NBUF
            pltpu.async_copy(idx_src(j), ibufs[b], lsems[b])
            pltpu.async_copy(rows_src(j), rbufs[b], lsems[b])

        def wait_loads(j):
            b = j % _NBUF
            pltpu.make_async_copy(idx_src(j), ibufs[b], lsems[b]).wait()
            pltpu.make_async_copy(rows_src(j), rbufs[b], lsems[b]).wait()

        def drain_scatter(j):
            b = j % _NBUF
            pltpu.make_async_copy(rows_src(j), rbufs[b], ssems[b]).wait()

        for j in range(_LEAD):
            start_loads(j)
        for j in range(_NCH):
            b = j % _NBUF
            wait_loads(j)
            pltpu.async_copy(
                rbufs[b], seg_sh.at[ibufs[b]], ssems[b], add=True
            )
            if j >= _LEAD:
                drain_scatter(j - _LEAD)
            if j + _LEAD < _NCH:
                start_loads(j + _LEAD)
        for j in range(_NCH - _LEAD, _NCH):
            drain_scatter(j)
        plsc.subcore_barrier()
        pltpu.sync_copy(
            seg_sh.at[pl.ds(s * _ZR, _ZR)], out_hbm.at[c, pl.ds(s * _ZR, _ZR)]
        )

    return seg_sum


def kernel(x, idx, W1p, b1p, W2p, b2p, W1r, b1r, W2r, b2r):
    idx32 = idx.astype(jnp.int32)
    b1p2, b2p2 = b1p.reshape(1, _H), b2p.reshape(1, _D)
    zrows = jnp.zeros((_ZR, _D), jnp.float32)
    seg_sum = _make_seg_sum()
    partials = []
    for t in range(_NSLAB):
        xs = lax.slice_in_dim(x, t * _NSR, (t + 1) * _NSR, axis=0)
        ids = lax.slice_in_dim(idx32, t * _NSR, (t + 1) * _NSR, axis=0)
        pt = _point_net(xs, W1p, b1p2, W2p, b2p2)
        partials.append(seg_sum(pt, ids, zrows))
    return _reduce_net(
        partials[0], partials[1], W1r, b1r.reshape(1, _H), W2r, b2r.reshape(1, _D)
    )


# speedup vs baseline: 1.4321x; 1.4321x over previous
import functools

import jax
import jax.numpy as jnp
from jax import lax
from jax.experimental import pallas as pl
from jax.experimental.pallas import tpu as pltpu
from jax.experimental.pallas import tpu_sc as plsc

_N = 320000
_D = 128
_H = 256
_S = 10000


_BN = 4000


def _pn_body(x_ref, w1_ref, b1_ref, w2_ref, b2_ref, o_ref):
    h = jnp.maximum(
        jnp.dot(x_ref[...], w1_ref[...], preferred_element_type=jnp.float32)
        + b1_ref[...],
        0.0,
    )
    o_ref[...] = (
        jnp.dot(h, w2_ref[...], preferred_element_type=jnp.float32) + b2_ref[...]
    )


def _point_net(x, w1, b1, w2, b2):
    return pl.pallas_call(
        _pn_body,
        grid=(_N // _BN,),
        in_specs=[
            pl.BlockSpec((_BN, _D), lambda i: (i, 0)),
            pl.BlockSpec((_D, _H), lambda i: (0, 0)),
            pl.BlockSpec((1, _H), lambda i: (0, 0)),
            pl.BlockSpec((_H, _D), lambda i: (0, 0)),
            pl.BlockSpec((1, _D), lambda i: (0, 0)),
        ],
        out_specs=pl.BlockSpec((_BN, _D), lambda i: (i, 0)),
        out_shape=jax.ShapeDtypeStruct((_N, _D), jnp.float32),
    )(x, w1, b1, w2, b2)


_BS = 2000


def _rn_body(p_ref, w1_ref, b1_ref, w2_ref, b2_ref, o_ref):
    seg = p_ref[0] + p_ref[1]
    h = jnp.maximum(
        jnp.dot(seg, w1_ref[...], preferred_element_type=jnp.float32) + b1_ref[...],
        0.0,
    )
    o_ref[...] = (
        jnp.dot(h, w2_ref[...], preferred_element_type=jnp.float32) + b2_ref[...]
    )


def _reduce_net(partials, w1, b1, w2, b2):
    return pl.pallas_call(
        _rn_body,
        grid=(_S // _BS,),
        in_specs=[
            pl.BlockSpec((2, _BS, _D), lambda i: (0, i, 0)),
            pl.BlockSpec((_D, _H), lambda i: (0, 0)),
            pl.BlockSpec((1, _H), lambda i: (0, 0)),
            pl.BlockSpec((_H, _D), lambda i: (0, 0)),
            pl.BlockSpec((1, _D), lambda i: (0, 0)),
        ],
        out_specs=pl.BlockSpec((_BS, _D), lambda i: (i, 0)),
        out_shape=jax.ShapeDtypeStruct((_S, _D), jnp.float32),
    )(partials, w1, b1, w2, b2)


_NC = 2
_NS = 16
_NW = _NC * _NS
_RPW = _N // _NW
_CH = 80
_NCH = _RPW // _CH
_NBUF = 4
_LEAD = 2
_IGRP = 5
_NGRP = _NCH // _IGRP
_SP = 10112
_ZR = _SP // _NS


@functools.cache
def _make_seg_sum():
    mesh = plsc.VectorSubcoreMesh(core_axis_name="c", subcore_axis_name="s")

    @functools.partial(
        pl.kernel,
        mesh=mesh,
        out_type=jax.ShapeDtypeStruct((_NC, _SP, _D), jnp.float32),
        scratch_types=(
            [pltpu.VMEM((_IGRP, _CH), jnp.int32) for _ in range(2)]
            + [pltpu.VMEM((_CH, _D), jnp.float32) for _ in range(_NBUF)]
            + [pltpu.VMEM_SHARED((_SP, _D), jnp.float32)]
            + [pltpu.SemaphoreType.DMA for _ in range(2 + 2 * _NBUF)]
        ),
    )
    def seg_sum(pt_hbm, idx4_hbm, zrows_hbm, out_hbm, *scr):
        igbufs = scr[:2]
        rbufs = scr[2 : 2 + _NBUF]
        seg_sh = scr[2 + _NBUF]
        isems = scr[3 + _NBUF : 5 + _NBUF]
        lsems = scr[5 + _NBUF : 5 + 2 * _NBUF]
        ssems = scr[5 + 2 * _NBUF : 5 + 3 * _NBUF]
        c = lax.axis_index("c")
        s = lax.axis_index("s")
        wid = c * _NS + s
        pltpu.sync_copy(zrows_hbm, seg_sh.at[pl.ds(s * _ZR, _ZR)])
        plsc.subcore_barrier()

        base = wid * _RPW

        def rows_src(j):
            return pt_hbm.at[pl.ds(base + j * _CH, _CH)]

        def idx_src(g):
            return idx4_hbm.at[wid, g]

        def start_idx(g):
            pltpu.async_copy(idx_src(g), igbufs[g % 2], isems[g % 2])

        def wait_idx(g):
            pltpu.make_async_copy(idx_src(g), igbufs[g % 2], isems[g % 2]).wait()

        def start_rows(j):
            b = j % _NBUF
            pltpu.async_copy(rows_src(j), rbufs[b], lsems[b])

        def wait_rows(j):
            b = j % _NBUF
            pltpu.make_async_copy(rows_src(j), rbufs[b], lsems[b]).wait()

        def drain_scatter(j):
            b = j % _NBUF
            pltpu.make_async_copy(rows_src(j), rbufs[b], ssems[b]).wait()

        start_idx(0)
        for j in range(_LEAD):
            start_rows(j)
        for j in range(_NCH):
            b = j % _NBUF
            g, k = divmod(j, _IGRP)
            if k == 0:
                wait_idx(g)
            wait_rows(j)
            pltpu.async_copy(
                rbufs[b], seg_sh.at[igbufs[g % 2].at[k]], ssems[b], add=True
            )
            if j >= _LEAD:
                drain_scatter(j - _LEAD)
            if k == 2 and g + 1 < _NGRP:
                start_idx(g + 1)
            if j + _LEAD < _NCH:
                start_rows(j + _LEAD)
        for j in range(_NCH - _LEAD, _NCH):
            drain_scatter(j)
        plsc.subcore_barrier()
        pltpu.sync_copy(
            seg_sh.at[pl.ds(s * _ZR, _ZR)], out_hbm.at[c, pl.ds(s * _ZR, _ZR)]
        )

    return seg_sum


def kernel(x, idx, W1p, b1p, W2p, b2p, W1r, b1r, W2r, b2r):
    idx4 = idx.astype(jnp.int32).reshape(_NW, _NGRP, _IGRP, _CH)
    pt = _point_net(x, W1p, b1p.reshape(1, _H), W2p, b2p.reshape(1, _D))
    zrows = jnp.zeros((_ZR, _D), jnp.float32)
    partials = _make_seg_sum()(pt, idx4, zrows)
    return _reduce_net(
        partials, W1r, b1r.reshape(1, _H), W2r, b2r.reshape(1, _D)
    )
